# Initial kernel scaffold; baseline (speedup 1.0000x reference)
import jax
import jax.numpy as jnp
from jax.experimental import pallas as pl


def kernel(x):
    raise NotImplementedError("write your pallas kernel here")


# speedup vs baseline: 1.0385x; 1.0385x over previous
import functools

import jax
import jax.numpy as jnp
from jax.experimental import pallas as pl
from jax.experimental.pallas import tpu as pltpu


def _avgmax_rows_kernel(x_ref, o_ref, *, inv_hw):
    x = x_ref[...]
    s = jnp.sum(x, axis=-1, keepdims=True)
    m = jnp.max(x, axis=-1, keepdims=True)
    o_ref[...] = (s * inv_hw + m).astype(o_ref.dtype)


def kernel(x):
    N, C, H, W = x.shape
    rows = N * C
    hw = H * W
    x2d = x.reshape(rows, hw)

    itemsize = jnp.dtype(x.dtype).itemsize
    sublane_mult = max(8, 32 // itemsize)

    row_tile = 512
    while row_tile > sublane_mult and rows % row_tile != 0:
        row_tile //= 2
    row_tile = max(sublane_mult, min(row_tile, rows))
    num_blocks = pl.cdiv(rows, row_tile)

    out = pl.pallas_call(
        functools.partial(_avgmax_rows_kernel, inv_hw=1.0 / hw),
        out_shape=jax.ShapeDtypeStruct((rows, 1), x.dtype),
        grid=(num_blocks,),
        in_specs=[pl.BlockSpec((row_tile, hw), lambda i: (i, 0))],
        out_specs=pl.BlockSpec((row_tile, 1), lambda i: (i, 0)),
        compiler_params=pltpu.CompilerParams(
            dimension_semantics=("parallel",),
            vmem_limit_bytes=64 * 1024 * 1024,
        ),
    )(x2d)

    return out.reshape(N, C, 1, 1)


# speedup vs baseline: 1.3213x; 1.2723x over previous
import functools

import jax
import jax.numpy as jnp
from jax.experimental import pallas as pl
from jax.experimental.pallas import tpu as pltpu


def _avgmax_nchw_kernel(x_ref, o_ref, *, inv_hw):
    x = x_ref[0]
    s1 = jnp.sum(x, axis=1)
    m1 = jnp.max(x, axis=1)
    s = jnp.sum(s1, axis=-1, keepdims=True)
    m = jnp.max(m1, axis=-1, keepdims=True)
    o_ref[...] = (s * inv_hw + m).astype(o_ref.dtype)


def kernel(x):
    N, C, H, W = x.shape
    rows = N * C
    hw = H * W

    out = pl.pallas_call(
        functools.partial(_avgmax_nchw_kernel, inv_hw=1.0 / hw),
        out_shape=jax.ShapeDtypeStruct((rows, 1), x.dtype),
        grid=(N,),
        in_specs=[pl.BlockSpec((1, C, H, W), lambda i: (i, 0, 0, 0))],
        out_specs=pl.BlockSpec((C, 1), lambda i: (i, 0)),
        compiler_params=pltpu.CompilerParams(
            dimension_semantics=("parallel",),
            vmem_limit_bytes=64 * 1024 * 1024,
        ),
    )(x)

    return out.reshape(N, C, 1, 1)


# speedup vs baseline: 9.7564x; 7.3841x over previous
import functools

import jax
import jax.numpy as jnp
from jax.experimental import pallas as pl
from jax.experimental.pallas import tpu as pltpu


def _avgmax_nhwc_kernel(x_ref, o_ref, *, inv_hw):
    x = x_ref[...]
    s = jnp.sum(jnp.sum(x, axis=1), axis=1)
    m = jnp.max(jnp.max(x, axis=1), axis=1)
    o_ref[0] = (s * inv_hw + m).astype(o_ref.dtype)


def kernel(x):
    N, C, H, W = x.shape
    hw = H * W

    x_t = jnp.transpose(x, (0, 2, 3, 1))

    batch_tile = 4
    while batch_tile > 1 and N % batch_tile != 0:
        batch_tile //= 2
    num_blocks = N // batch_tile

    out = pl.pallas_call(
        functools.partial(_avgmax_nhwc_kernel, inv_hw=1.0 / hw),
        out_shape=jax.ShapeDtypeStruct((num_blocks, batch_tile, C), x.dtype),
        grid=(num_blocks,),
        in_specs=[pl.BlockSpec((batch_tile, H, W, C), lambda i: (i, 0, 0, 0))],
        out_specs=pl.BlockSpec((1, batch_tile, C), lambda i: (i, 0, 0)),
        compiler_params=pltpu.CompilerParams(
            dimension_semantics=("parallel",),
            vmem_limit_bytes=64 * 1024 * 1024,
        ),
    )(x_t)

    return out.reshape(N, C, 1, 1)
